# baseline (speedup 1.0000x reference)
import jax
import jax.numpy as jnp
from jax.experimental import pallas as pl

N = 10000


def _nnconv(h, edge_index, edge_attr, w1, b1, w2, b2, root, bias, in_ch, out_ch):
    e = jax.nn.relu(edge_attr @ w1 + b1)
    w = (e @ w2 + b2).reshape(-1, in_ch, out_ch)
    src = edge_index[0]
    dst = edge_index[1]
    msg = jnp.einsum('ei,eio->eo', h[src], w)
    summed = jax.ops.segment_sum(msg, dst, num_segments=N)
    cnt = jax.ops.segment_sum(jnp.ones((msg.shape[0],), msg.dtype), dst, num_segments=N)
    mean = summed / jnp.maximum(cnt, 1.0)[:, None]
    return mean + h @ root + bias


def _id_kernel(x_ref, o_ref):
    o_ref[...] = x_ref[...]


def kernel(x, t, edge_index, edge_attr, y, fc1_w, fc1_b, fc2_w, fc2_b, nn1_w1, nn1_b1, nn1_w2, nn1_b2, conv1_root, conv1_bias, nn3_w1, nn3_b1, nn3_w2, nn3_b2, conv3_root, conv3_bias, fc3_w, fc3_b, fc4_w, fc4_b):
    T = t.shape[0]
    boundary = y[0, :].reshape(-1, 1)
    res = [boundary.reshape(-1)]
    h0 = jax.nn.relu(boundary @ fc1_w + fc1_b)
    h = jax.nn.relu(h0 @ fc2_w + fc2_b)
    x_i = jnp.tile(x, (1, 3))
    for i in range(1, T):
        t_i = jnp.broadcast_to(t[i], (N, 3))
        h = jnp.concatenate([h, x_i, t_i], axis=1)
        h = jax.nn.relu(_nnconv(h, edge_index, edge_attr, nn1_w1, nn1_b1, nn1_w2, nn1_b2, conv1_root, conv1_bias, 32, 32))
        h = jax.nn.relu(_nnconv(h, edge_index, edge_attr, nn3_w1, nn3_b1, nn3_w2, nn3_b2, conv3_root, conv3_bias, 32, 26))
        yy = jax.nn.relu(h @ fc3_w + fc3_b)
        yy = yy @ fc4_w + fc4_b
        res.append(yy.reshape(-1))
    out = jnp.stack(res, axis=0).reshape(-1)
    return pl.pallas_call(
        _id_kernel, out_shape=jax.ShapeDtypeStruct(out.shape, out.dtype)
    )(out)


# speedup vs baseline: 1.5154x; 1.5154x over previous
import functools

import jax
import jax.numpy as jnp
from jax import lax
from jax.experimental import pallas as pl
from jax.experimental.pallas import tpu as pltpu
from jax.experimental.pallas import tpu_sc as plsc

N = 10000
E = 160000
NP = 10016
EP = 163840
NWORK = 32
EPW = EP // NWORK
CH = 128
NCHUNK = EPW // CH
ROWS_PER_SUB = NP // 16
BE = 2048
GRID_E = EP // BE

_mesh = plsc.VectorSubcoreMesh(core_axis_name="c", subcore_axis_name="s")


@functools.partial(
    pl.kernel, mesh=_mesh,
    out_type=jax.ShapeDtypeStruct((EP, 32), jnp.float32),
    compiler_params=pltpu.CompilerParams(use_tc_tiling_on_sc=False),
    scratch_types=[
        pltpu.VMEM((NCHUNK, CH), jnp.int32),
        pltpu.VMEM((CH, 32), jnp.float32),
        pltpu.SemaphoreType.DMA,
    ],
)
def _sc_gather(table_hbm, src2d_hbm, out_hbm, idx_v, rows_v, sem):
    c = lax.axis_index("c")
    s = lax.axis_index("s")
    wid = c * 16 + s
    pltpu.sync_copy(src2d_hbm.at[pl.ds(wid * NCHUNK, NCHUNK)], idx_v)
    base = wid * EPW

    def body(j, carry):
        pltpu.async_copy(table_hbm.at[idx_v.at[j]], rows_v, sem).wait()
        pltpu.sync_copy(rows_v, out_hbm.at[pl.ds(base + j * CH, CH)])
        return carry

    lax.fori_loop(0, NCHUNK, body, 0)


@functools.partial(
    pl.kernel, mesh=_mesh,
    out_type=jax.ShapeDtypeStruct((2, NP, 32), jnp.float32),
    compiler_params=pltpu.CompilerParams(use_tc_tiling_on_sc=False),
    scratch_types=[
        pltpu.VMEM((NCHUNK, CH), jnp.int32),
        pltpu.VMEM((CH, 32), jnp.float32),
        pltpu.VMEM_SHARED((NP, 32), jnp.float32),
    ],
)
def _sc_scatter(msg_hbm, dst2d_hbm, zeros_hbm, out_hbm, idx_v, rows_v, acc_sh):
    c = lax.axis_index("c")
    s = lax.axis_index("s")
    wid = c * 16 + s
    pltpu.sync_copy(zeros_hbm.at[pl.ds(s * ROWS_PER_SUB, ROWS_PER_SUB)],
                    acc_sh.at[pl.ds(s * ROWS_PER_SUB, ROWS_PER_SUB)])
    pltpu.sync_copy(dst2d_hbm.at[pl.ds(wid * NCHUNK, NCHUNK)], idx_v)
    plsc.subcore_barrier()
    base = wid * EPW

    def body(j, carry):
        pltpu.sync_copy(msg_hbm.at[pl.ds(base + j * CH, CH)], rows_v)
        pltpu.sync_copy(rows_v, acc_sh.at[idx_v.at[j]], add=True)
        return carry

    lax.fori_loop(0, NCHUNK, body, 0)
    plsc.subcore_barrier()
    pltpu.sync_copy(acc_sh.at[pl.ds(s * ROWS_PER_SUB, ROWS_PER_SUB)],
                    out_hbm.at[c].at[pl.ds(s * ROWS_PER_SUB, ROWS_PER_SUB)])


def _edge_mlp_body(attr_ref, w1a_ref, b1a_ref, w1b_ref, b1b_ref, e1_ref, e3_ref):
    a = attr_ref[...]
    e1_ref[...] = jnp.maximum(a * w1a_ref[...] + b1a_ref[...], 0.0)
    e3_ref[...] = jnp.maximum(a * w1b_ref[...] + b1b_ref[...], 0.0)


def _edge_mlp(attrp, w1a, b1a, w1b, b1b):
    return pl.pallas_call(
        _edge_mlp_body,
        grid=(GRID_E,),
        in_specs=[
            pl.BlockSpec((BE, 1), lambda j: (j, 0)),
            pl.BlockSpec((1, 16), lambda j: (0, 0)),
            pl.BlockSpec((1, 16), lambda j: (0, 0)),
            pl.BlockSpec((1, 16), lambda j: (0, 0)),
            pl.BlockSpec((1, 16), lambda j: (0, 0)),
        ],
        out_specs=[
            pl.BlockSpec((BE, 16), lambda j: (j, 0)),
            pl.BlockSpec((BE, 16), lambda j: (j, 0)),
        ],
        out_shape=[
            jax.ShapeDtypeStruct((EP, 16), jnp.float32),
            jax.ShapeDtypeStruct((EP, 16), jnp.float32),
        ],
    )(attrp, w1a, b1a, w1b, b1b)


def _msg_body(hs_ref, e_ref, a_ref, bm_ref, out_ref):
    hs = hs_ref[...]
    g = jnp.dot(hs, a_ref[...], preferred_element_type=jnp.float32)
    acc = jnp.dot(hs, bm_ref[...], preferred_element_type=jnp.float32)
    e = e_ref[...]
    for k in range(16):
        acc = acc + e[:, k:k + 1] * g[:, 32 * k:32 * (k + 1)]
    out_ref[...] = acc


def _msg(hsrc, eact, amat, bmat):
    return pl.pallas_call(
        _msg_body,
        grid=(GRID_E,),
        in_specs=[
            pl.BlockSpec((BE, 32), lambda j: (j, 0)),
            pl.BlockSpec((BE, 16), lambda j: (j, 0)),
            pl.BlockSpec((32, 512), lambda j: (0, 0)),
            pl.BlockSpec((32, 32), lambda j: (0, 0)),
        ],
        out_specs=pl.BlockSpec((BE, 32), lambda j: (j, 0)),
        out_shape=jax.ShapeDtypeStruct((EP, 32), jnp.float32),
    )(hsrc, eact, amat, bmat)


def _prologue_body(bnd_ref, w1_ref, b1_ref, w2_ref, b2_ref, xt_ref, out_ref):
    h0 = jnp.maximum(bnd_ref[...] * w1_ref[...] + b1_ref[...], 0.0)
    h26 = jnp.maximum(
        jnp.dot(h0, w2_ref[...], preferred_element_type=jnp.float32)
        + b2_ref[...], 0.0)
    out_ref[...] = h26 + xt_ref[...]


def _prologue(bnd, fc1_w, fc1_b, fc2p, fc2bp, xt1):
    return pl.pallas_call(
        _prologue_body,
        out_shape=jax.ShapeDtypeStruct((NP, 32), jnp.float32),
    )(bnd, fc1_w, fc1_b, fc2p, fc2bp, xt1)


def _fin1_body(p_ref, cp_ref, h32_ref, root_ref, bias_ref, out_ref):
    cnt = cp_ref[0][:, 0:1] + cp_ref[1][:, 0:1]
    invc = 1.0 / jnp.maximum(cnt, 1.0)
    mean = (p_ref[0] + p_ref[1]) * invc
    out_ref[...] = jnp.maximum(
        mean + jnp.dot(h32_ref[...], root_ref[...],
                       preferred_element_type=jnp.float32) + bias_ref[...], 0.0)


def _fin1(p, cp, h32, root, bias):
    return pl.pallas_call(
        _fin1_body,
        out_shape=jax.ShapeDtypeStruct((NP, 32), jnp.float32),
    )(p, cp, h32, root, bias)


def _fin3_body(p_ref, cp_ref, hl1_ref, root_ref, bias_ref, fc3_ref, fc3b_ref,
               fc4_ref, fc4b_ref, xt_ref, h32_ref, yy_ref):
    cnt = cp_ref[0][:, 0:1] + cp_ref[1][:, 0:1]
    invc = 1.0 / jnp.maximum(cnt, 1.0)
    mean = (p_ref[0] + p_ref[1]) * invc
    h26 = jnp.maximum(
        mean + jnp.dot(hl1_ref[...], root_ref[...],
                       preferred_element_type=jnp.float32) + bias_ref[...], 0.0)
    z = jnp.maximum(
        jnp.dot(h26, fc3_ref[...], preferred_element_type=jnp.float32)
        + fc3b_ref[...], 0.0)
    yy_ref[...] = (jnp.dot(z, fc4_ref[...], preferred_element_type=jnp.float32)
                   + fc4b_ref[...])
    h32_ref[...] = h26 + xt_ref[...]


def _fin3(p, cp, hl1, root, bias, fc3p, fc3b, fc4, fc4b, xt):
    return pl.pallas_call(
        _fin3_body,
        out_shape=[
            jax.ShapeDtypeStruct((NP, 32), jnp.float32),
            jax.ShapeDtypeStruct((NP, 1), jnp.float32),
        ],
    )(p, cp, hl1, root, bias, fc3p, fc3b, fc4, fc4b, xt)


def kernel(x, t, edge_index, edge_attr, y, fc1_w, fc1_b, fc2_w, fc2_b,
           nn1_w1, nn1_b1, nn1_w2, nn1_b2, conv1_root, conv1_bias,
           nn3_w1, nn3_b1, nn3_w2, nn3_b2, conv3_root, conv3_bias,
           fc3_w, fc3_b, fc4_w, fc4_b):
    f32 = jnp.float32
    pad_e = EP - E

    src = edge_index[0].astype(jnp.int32)
    dst = edge_index[1].astype(jnp.int32)
    src2d = jnp.concatenate([src, jnp.zeros((pad_e,), jnp.int32)]).reshape(-1, CH)
    dst2d = jnp.concatenate(
        [dst, jnp.full((pad_e,), N, jnp.int32)]).reshape(-1, CH)
    attrp = jnp.pad(edge_attr, ((0, pad_e), (0, 0)))

    a1 = nn1_w2.reshape(16, 32, 32).transpose(1, 0, 2).reshape(32, 512)
    bm1 = nn1_b2.reshape(32, 32)
    a3 = jnp.pad(nn3_w2.reshape(16, 32, 26),
                 ((0, 0), (0, 0), (0, 6))).transpose(1, 0, 2).reshape(32, 512)
    bm3 = jnp.pad(nn3_b2.reshape(32, 26), ((0, 0), (0, 6)))
    root1 = conv1_root
    bias1 = conv1_bias.reshape(1, 32)
    root3 = jnp.pad(conv3_root, ((0, 0), (0, 6)))
    bias3 = jnp.pad(conv3_bias, (0, 6)).reshape(1, 32)
    fc2p = jnp.pad(fc2_w, ((0, 0), (0, 6)))
    fc2bp = jnp.pad(fc2_b, (0, 6)).reshape(1, 32)
    fc3p = jnp.pad(fc3_w, ((0, 6), (0, 0)))
    fc3b = fc3_b.reshape(1, 32)
    fc4b = fc4_b.reshape(1, 1)

    xp = jnp.pad(x, ((0, NP - N), (0, 0)))
    zeros26 = jnp.zeros((NP, 26), f32)

    def xt_for(ti):
        return jnp.concatenate(
            [zeros26, xp, xp, xp, jnp.broadcast_to(ti, (NP, 3))], axis=1)

    zeros_np = jnp.zeros((NP, 32), f32)
    ones_ep = jnp.ones((EP, 32), f32)

    e1, e3 = _edge_mlp(attrp, nn1_w1, nn1_b1.reshape(1, 16),
                       nn3_w1, nn3_b1.reshape(1, 16))
    cp = _sc_scatter(ones_ep, dst2d, zeros_np)

    bnd = jnp.pad(y[0].reshape(-1, 1), ((0, NP - N), (0, 0)))
    h32 = _prologue(bnd, fc1_w, fc1_b.reshape(1, 32), fc2p, fc2bp, xt_for(t[1]))

    ys = []
    T = t.shape[0]
    for i in range(1, T):
        hs1 = _sc_gather(h32, src2d)
        m1 = _msg(hs1, e1, a1, bm1)
        p1 = _sc_scatter(m1, dst2d, zeros_np)
        hl1 = _fin1(p1, cp, h32, root1, bias1)

        hs3 = _sc_gather(hl1, src2d)
        m3 = _msg(hs3, e3, a3, bm3)
        p3 = _sc_scatter(m3, dst2d, zeros_np)
        xt_next = xt_for(t[i + 1]) if i + 1 < T else zeros_np
        h32, yy = _fin3(p3, cp, hl1, root3, bias3, fc3p, fc3b, fc4_w, fc4b,
                        xt_next)
        ys.append(yy[:N, 0])

    return jnp.concatenate([y[0]] + ys)


# speedup vs baseline: 3.1965x; 2.1094x over previous
import functools

import jax
import jax.numpy as jnp
from jax import lax
from jax.experimental import pallas as pl
from jax.experimental.pallas import tpu as pltpu
from jax.experimental.pallas import tpu_sc as plsc

N = 10000
E = 160000
NP = 10016
EP = 163840
NWORK = 32
EPW = EP // NWORK
CH = 128
NCHUNK = EPW // CH
ROWS_PER_SUB = NP // 16
BE = 2048
GRID_E = EP // BE

_mesh = plsc.VectorSubcoreMesh(core_axis_name="c", subcore_axis_name="s")


@functools.partial(
    pl.kernel, mesh=_mesh,
    out_type=jax.ShapeDtypeStruct((EP, 32), jnp.float32),
    compiler_params=pltpu.CompilerParams(use_tc_tiling_on_sc=False),
    scratch_types=[
        pltpu.VMEM((NCHUNK, CH), jnp.int32),
        pltpu.VMEM((CH, 32), jnp.float32),
        pltpu.SemaphoreType.DMA,
    ],
)
def _sc_gather(table_hbm, src2d_hbm, out_hbm, idx_v, rows_v, sem):
    c = lax.axis_index("c")
    s = lax.axis_index("s")
    wid = c * 16 + s
    pltpu.sync_copy(src2d_hbm.at[pl.ds(wid * NCHUNK, NCHUNK)], idx_v)
    base = wid * EPW

    def body(j, carry):
        pltpu.async_copy(table_hbm.at[idx_v.at[j]], rows_v, sem).wait()
        pltpu.sync_copy(rows_v, out_hbm.at[pl.ds(base + j * CH, CH)])
        return carry

    lax.fori_loop(0, NCHUNK, body, 0)


@functools.partial(
    pl.kernel, mesh=_mesh,
    out_type=jax.ShapeDtypeStruct((2, NP, 32), jnp.float32),
    compiler_params=pltpu.CompilerParams(use_tc_tiling_on_sc=False),
    scratch_types=[
        pltpu.VMEM((NCHUNK, CH), jnp.int32),
        pltpu.VMEM((CH, 32), jnp.float32),
        pltpu.VMEM_SHARED((NP, 32), jnp.float32),
    ],
)
def _sc_scatter(msg_hbm, dst2d_hbm, zeros_hbm, out_hbm, idx_v, rows_v, acc_sh):
    c = lax.axis_index("c")
    s = lax.axis_index("s")
    wid = c * 16 + s
    pltpu.sync_copy(zeros_hbm.at[pl.ds(s * ROWS_PER_SUB, ROWS_PER_SUB)],
                    acc_sh.at[pl.ds(s * ROWS_PER_SUB, ROWS_PER_SUB)])
    pltpu.sync_copy(dst2d_hbm.at[pl.ds(wid * NCHUNK, NCHUNK)], idx_v)
    plsc.subcore_barrier()
    base = wid * EPW

    def body(j, carry):
        pltpu.sync_copy(msg_hbm.at[pl.ds(base + j * CH, CH)], rows_v)
        pltpu.sync_copy(rows_v, acc_sh.at[idx_v.at[j]], add=True)
        return carry

    lax.fori_loop(0, NCHUNK, body, 0)
    plsc.subcore_barrier()
    pltpu.sync_copy(acc_sh.at[pl.ds(s * ROWS_PER_SUB, ROWS_PER_SUB)],
                    out_hbm.at[c].at[pl.ds(s * ROWS_PER_SUB, ROWS_PER_SUB)])


@functools.partial(
    pl.kernel, mesh=_mesh,
    out_type=jax.ShapeDtypeStruct((2, NP, 32), jnp.float32),
    compiler_params=pltpu.CompilerParams(use_tc_tiling_on_sc=False),
    scratch_types=[
        pltpu.VMEM((NCHUNK, CH), jnp.int32),
        pltpu.VMEM((CH, 32), jnp.float32),
        pltpu.VMEM_SHARED((NP, 32), jnp.float32),
    ],
)
def _sc_count(ones_hbm, dst2d_hbm, zeros_hbm, out_hbm, idx_v, rows_v, acc_sh):
    c = lax.axis_index("c")
    s = lax.axis_index("s")
    wid = c * 16 + s
    pltpu.sync_copy(zeros_hbm.at[pl.ds(s * ROWS_PER_SUB, ROWS_PER_SUB)],
                    acc_sh.at[pl.ds(s * ROWS_PER_SUB, ROWS_PER_SUB)])
    pltpu.sync_copy(dst2d_hbm.at[pl.ds(wid * NCHUNK, NCHUNK)], idx_v)
    pltpu.sync_copy(ones_hbm, rows_v)
    plsc.subcore_barrier()

    def body(j, carry):
        pltpu.sync_copy(rows_v, acc_sh.at[idx_v.at[j]], add=True)
        return carry

    lax.fori_loop(0, NCHUNK, body, 0)
    plsc.subcore_barrier()
    pltpu.sync_copy(acc_sh.at[pl.ds(s * ROWS_PER_SUB, ROWS_PER_SUB)],
                    out_hbm.at[c].at[pl.ds(s * ROWS_PER_SUB, ROWS_PER_SUB)])


def _msg_body(hs_ref, attr_ref, w1_ref, b1_ref, t_ref, w2s_ref, bm_ref,
              out_ref):
    hs = hs_ref[...]
    hsb = hs.astype(jnp.bfloat16)
    e = jnp.maximum(attr_ref[...] * w1_ref[...] + b1_ref[...], 0.0)
    e_tile = pltpu.repeat(e.astype(jnp.bfloat16), 32, axis=1)
    h_exp = jnp.dot(hsb, t_ref[...],
                    preferred_element_type=jnp.float32).astype(jnp.bfloat16)
    u = h_exp * e_tile
    out_ref[...] = (
        jnp.dot(u, w2s_ref[...], preferred_element_type=jnp.float32)
        + jnp.dot(hs, bm_ref[...], preferred_element_type=jnp.float32))


def _msg(hsrc, attrp, w1, b1, tmat, w2s, bmat):
    return pl.pallas_call(
        _msg_body,
        grid=(GRID_E,),
        in_specs=[
            pl.BlockSpec((BE, 32), lambda j: (j, 0)),
            pl.BlockSpec((BE, 1), lambda j: (j, 0)),
            pl.BlockSpec((1, 16), lambda j: (0, 0)),
            pl.BlockSpec((1, 16), lambda j: (0, 0)),
            pl.BlockSpec((32, 512), lambda j: (0, 0)),
            pl.BlockSpec((512, 32), lambda j: (0, 0)),
            pl.BlockSpec((32, 32), lambda j: (0, 0)),
        ],
        out_specs=pl.BlockSpec((BE, 32), lambda j: (j, 0)),
        out_shape=jax.ShapeDtypeStruct((EP, 32), jnp.float32),
    )(hsrc, attrp, w1, b1, tmat, w2s, bmat)


def _prologue_body(bnd_ref, w1_ref, b1_ref, w2_ref, b2_ref, xt_ref, out_ref):
    h0 = jnp.maximum(bnd_ref[...] * w1_ref[...] + b1_ref[...], 0.0)
    h26 = jnp.maximum(
        jnp.dot(h0, w2_ref[...], preferred_element_type=jnp.float32)
        + b2_ref[...], 0.0)
    out_ref[...] = h26 + xt_ref[...]


def _prologue(bnd, fc1_w, fc1_b, fc2p, fc2bp, xt1):
    return pl.pallas_call(
        _prologue_body,
        out_shape=jax.ShapeDtypeStruct((NP, 32), jnp.float32),
    )(bnd, fc1_w, fc1_b, fc2p, fc2bp, xt1)


def _fin1_body(p_ref, cp_ref, h32_ref, root_ref, bias_ref, out_ref):
    cnt = cp_ref[0][:, 0:1] + cp_ref[1][:, 0:1]
    invc = 1.0 / jnp.maximum(cnt, 1.0)
    mean = (p_ref[0] + p_ref[1]) * invc
    out_ref[...] = jnp.maximum(
        mean + jnp.dot(h32_ref[...], root_ref[...],
                       preferred_element_type=jnp.float32) + bias_ref[...], 0.0)


def _fin1(p, cp, h32, root, bias):
    return pl.pallas_call(
        _fin1_body,
        out_shape=jax.ShapeDtypeStruct((NP, 32), jnp.float32),
    )(p, cp, h32, root, bias)


def _fin3_body(p_ref, cp_ref, hl1_ref, root_ref, bias_ref, fc3_ref, fc3b_ref,
               fc4_ref, fc4b_ref, xt_ref, h32_ref, yy_ref):
    cnt = cp_ref[0][:, 0:1] + cp_ref[1][:, 0:1]
    invc = 1.0 / jnp.maximum(cnt, 1.0)
    mean = (p_ref[0] + p_ref[1]) * invc
    h26 = jnp.maximum(
        mean + jnp.dot(hl1_ref[...], root_ref[...],
                       preferred_element_type=jnp.float32) + bias_ref[...], 0.0)
    z = jnp.maximum(
        jnp.dot(h26, fc3_ref[...], preferred_element_type=jnp.float32)
        + fc3b_ref[...], 0.0)
    yy_ref[...] = (jnp.dot(z, fc4_ref[...], preferred_element_type=jnp.float32)
                   + fc4b_ref[...])
    h32_ref[...] = h26 + xt_ref[...]


def _fin3(p, cp, hl1, root, bias, fc3p, fc3b, fc4, fc4b, xt):
    return pl.pallas_call(
        _fin3_body,
        out_shape=[
            jax.ShapeDtypeStruct((NP, 32), jnp.float32),
            jax.ShapeDtypeStruct((NP, 1), jnp.float32),
        ],
    )(p, cp, hl1, root, bias, fc3p, fc3b, fc4, fc4b, xt)


def kernel(x, t, edge_index, edge_attr, y, fc1_w, fc1_b, fc2_w, fc2_b,
           nn1_w1, nn1_b1, nn1_w2, nn1_b2, conv1_root, conv1_bias,
           nn3_w1, nn3_b1, nn3_w2, nn3_b2, conv3_root, conv3_bias,
           fc3_w, fc3_b, fc4_w, fc4_b):
    f32 = jnp.float32
    pad_e = EP - E

    src = edge_index[0].astype(jnp.int32)
    dst = edge_index[1].astype(jnp.int32)
    src2d = jnp.concatenate([src, jnp.zeros((pad_e,), jnp.int32)]).reshape(-1, CH)
    dst2d = jnp.concatenate(
        [dst, jnp.full((pad_e,), N, jnp.int32)]).reshape(-1, CH)
    attrp = jnp.pad(edge_attr, ((0, pad_e), (0, 0)))

    w2s1 = nn1_w2.reshape(16, 32, 32).transpose(1, 0, 2).reshape(512, 32)
    bm1 = nn1_b2.reshape(32, 32)
    w2s3 = jnp.pad(nn3_w2.reshape(16, 32, 26),
                   ((0, 0), (0, 0), (0, 6))).transpose(1, 0, 2).reshape(512, 32)
    bm3 = jnp.pad(nn3_b2.reshape(32, 26), ((0, 0), (0, 6)))
    tmat = jnp.kron(jnp.eye(32, dtype=jnp.bfloat16),
                    jnp.ones((1, 16), jnp.bfloat16))
    w2s1 = w2s1.astype(jnp.bfloat16)
    w2s3 = w2s3.astype(jnp.bfloat16)
    w1e1 = nn1_w1
    b1e1 = nn1_b1.reshape(1, 16)
    w1e3 = nn3_w1
    b1e3 = nn3_b1.reshape(1, 16)
    root1 = conv1_root
    bias1 = conv1_bias.reshape(1, 32)
    root3 = jnp.pad(conv3_root, ((0, 0), (0, 6)))
    bias3 = jnp.pad(conv3_bias, (0, 6)).reshape(1, 32)
    fc2p = jnp.pad(fc2_w, ((0, 0), (0, 6)))
    fc2bp = jnp.pad(fc2_b, (0, 6)).reshape(1, 32)
    fc3p = jnp.pad(fc3_w, ((0, 6), (0, 0)))
    fc3b = fc3_b.reshape(1, 32)
    fc4b = fc4_b.reshape(1, 1)

    xp = jnp.pad(x, ((0, NP - N), (0, 0)))
    zeros26 = jnp.zeros((NP, 26), f32)

    def xt_for(ti):
        return jnp.concatenate(
            [zeros26, xp, xp, xp, jnp.broadcast_to(ti, (NP, 3))], axis=1)

    zeros_np = jnp.zeros((NP, 32), f32)
    ones_ch = jnp.ones((CH, 32), f32)

    cp = _sc_count(ones_ch, dst2d, zeros_np)

    bnd = jnp.pad(y[0].reshape(-1, 1), ((0, NP - N), (0, 0)))
    h32 = _prologue(bnd, fc1_w, fc1_b.reshape(1, 32), fc2p, fc2bp, xt_for(t[1]))

    ys = []
    T = t.shape[0]
    for i in range(1, T):
        hs1 = _sc_gather(h32, src2d)
        m1 = _msg(hs1, attrp, w1e1, b1e1, tmat, w2s1, bm1)
        p1 = _sc_scatter(m1, dst2d, zeros_np)
        hl1 = _fin1(p1, cp, h32, root1, bias1)

        hs3 = _sc_gather(hl1, src2d)
        m3 = _msg(hs3, attrp, w1e3, b1e3, tmat, w2s3, bm3)
        p3 = _sc_scatter(m3, dst2d, zeros_np)
        xt_next = xt_for(t[i + 1]) if i + 1 < T else zeros_np
        h32, yy = _fin3(p3, cp, hl1, root3, bias3, fc3p, fc3b, fc4_w, fc4b,
                        xt_next)
        ys.append(yy[:N, 0])

    return jnp.concatenate([y[0]] + ys)


# speedup vs baseline: 3.5614x; 1.1141x over previous
import functools

import jax
import jax.numpy as jnp
from jax import lax
from jax.experimental import pallas as pl
from jax.experimental.pallas import tpu as pltpu
from jax.experimental.pallas import tpu_sc as plsc

N = 10000
E = 160000
NP = 10016
EP = 163840
NWORK = 32
EPW = EP // NWORK
CH = 128
NCHUNK = EPW // CH
ROWS_PER_SUB = NP // 16
BE = 2048
GRID_E = EP // BE

_mesh = plsc.VectorSubcoreMesh(core_axis_name="c", subcore_axis_name="s")


NBUF = 4


@functools.partial(
    pl.kernel, mesh=_mesh,
    out_type=jax.ShapeDtypeStruct((EP, 32), jnp.float32),
    compiler_params=pltpu.CompilerParams(use_tc_tiling_on_sc=False),
    scratch_types=[
        pltpu.VMEM((NCHUNK, CH), jnp.int32),
        pltpu.VMEM((NBUF * CH, 32), jnp.float32),
        pltpu.SemaphoreType.DMA((NBUF,)),
    ],
)
def _sc_gather(table_hbm, src2d_hbm, out_hbm, idx_v, rows_v, gsem):
    c = lax.axis_index("c")
    s = lax.axis_index("s")
    wid = c * 16 + s
    pltpu.sync_copy(src2d_hbm.at[pl.ds(wid * NCHUNK, NCHUNK)], idx_v)
    base = wid * EPW
    for b in range(NBUF):
        pltpu.async_copy(table_hbm.at[idx_v.at[b]],
                         rows_v.at[pl.ds(b * CH, CH)], gsem.at[b])

    def body(g, carry):
        for b in range(NBUF):
            j = g * NBUF + b
            buf = rows_v.at[pl.ds(b * CH, CH)]
            pltpu.make_async_copy(table_hbm.at[idx_v.at[j]], buf,
                                  gsem.at[b]).wait()
            pltpu.sync_copy(buf, out_hbm.at[pl.ds(base + j * CH, CH)])
            jn = j + NBUF

            @pl.when(jn < NCHUNK)
            def _():
                pltpu.async_copy(table_hbm.at[idx_v.at[jn]], buf, gsem.at[b])
        return carry

    lax.fori_loop(0, NCHUNK // NBUF, body, 0)


@functools.partial(
    pl.kernel, mesh=_mesh,
    out_type=jax.ShapeDtypeStruct((2, NP, 32), jnp.float32),
    compiler_params=pltpu.CompilerParams(use_tc_tiling_on_sc=False),
    scratch_types=[
        pltpu.VMEM((NCHUNK, CH), jnp.int32),
        pltpu.VMEM((NBUF * CH, 32), jnp.float32),
        pltpu.SemaphoreType.DMA((NBUF,)),
        pltpu.VMEM_SHARED((NP, 32), jnp.float32),
    ],
)
def _sc_scatter(msg_hbm, dst2d_hbm, zeros_hbm, out_hbm, idx_v, rows_v, lsem,
                acc_sh):
    c = lax.axis_index("c")
    s = lax.axis_index("s")
    wid = c * 16 + s
    pltpu.sync_copy(zeros_hbm.at[pl.ds(s * ROWS_PER_SUB, ROWS_PER_SUB)],
                    acc_sh.at[pl.ds(s * ROWS_PER_SUB, ROWS_PER_SUB)])
    pltpu.sync_copy(dst2d_hbm.at[pl.ds(wid * NCHUNK, NCHUNK)], idx_v)
    plsc.subcore_barrier()
    base = wid * EPW
    for b in range(NBUF):
        pltpu.async_copy(msg_hbm.at[pl.ds(base + b * CH, CH)],
                         rows_v.at[pl.ds(b * CH, CH)], lsem.at[b])

    def body(g, carry):
        for b in range(NBUF):
            j = g * NBUF + b
            buf = rows_v.at[pl.ds(b * CH, CH)]
            pltpu.make_async_copy(msg_hbm.at[pl.ds(base + j * CH, CH)], buf,
                                  lsem.at[b]).wait()
            pltpu.sync_copy(buf, acc_sh.at[idx_v.at[j]], add=True)
            jn = j + NBUF

            @pl.when(jn < NCHUNK)
            def _():
                pltpu.async_copy(msg_hbm.at[pl.ds(base + jn * CH, CH)], buf,
                                 lsem.at[b])
        return carry

    lax.fori_loop(0, NCHUNK // NBUF, body, 0)
    plsc.subcore_barrier()
    pltpu.sync_copy(acc_sh.at[pl.ds(s * ROWS_PER_SUB, ROWS_PER_SUB)],
                    out_hbm.at[c].at[pl.ds(s * ROWS_PER_SUB, ROWS_PER_SUB)])


@functools.partial(
    pl.kernel, mesh=_mesh,
    out_type=jax.ShapeDtypeStruct((2, NP, 32), jnp.float32),
    compiler_params=pltpu.CompilerParams(use_tc_tiling_on_sc=False),
    scratch_types=[
        pltpu.VMEM((NCHUNK, CH), jnp.int32),
        pltpu.VMEM((CH, 32), jnp.float32),
        pltpu.VMEM_SHARED((NP, 32), jnp.float32),
    ],
)
def _sc_count(ones_hbm, dst2d_hbm, zeros_hbm, out_hbm, idx_v, rows_v, acc_sh):
    c = lax.axis_index("c")
    s = lax.axis_index("s")
    wid = c * 16 + s
    pltpu.sync_copy(zeros_hbm.at[pl.ds(s * ROWS_PER_SUB, ROWS_PER_SUB)],
                    acc_sh.at[pl.ds(s * ROWS_PER_SUB, ROWS_PER_SUB)])
    pltpu.sync_copy(dst2d_hbm.at[pl.ds(wid * NCHUNK, NCHUNK)], idx_v)
    pltpu.sync_copy(ones_hbm, rows_v)
    plsc.subcore_barrier()

    def body(j, carry):
        pltpu.sync_copy(rows_v, acc_sh.at[idx_v.at[j]], add=True)
        return carry

    lax.fori_loop(0, NCHUNK, body, 0)
    plsc.subcore_barrier()
    pltpu.sync_copy(acc_sh.at[pl.ds(s * ROWS_PER_SUB, ROWS_PER_SUB)],
                    out_hbm.at[c].at[pl.ds(s * ROWS_PER_SUB, ROWS_PER_SUB)])


def _msg_body(hs_ref, attr_ref, w1_ref, b1_ref, t_ref, w2s_ref, bm_ref,
              out_ref):
    hs = hs_ref[...]
    hsb = hs.astype(jnp.bfloat16)
    e = jnp.maximum(attr_ref[...] * w1_ref[...] + b1_ref[...], 0.0)
    e_tile = pltpu.repeat(e.astype(jnp.bfloat16), 32, axis=1)
    h_exp = jnp.dot(hsb, t_ref[...],
                    preferred_element_type=jnp.float32).astype(jnp.bfloat16)
    u = h_exp * e_tile
    out_ref[...] = (
        jnp.dot(u, w2s_ref[...], preferred_element_type=jnp.float32)
        + jnp.dot(hs, bm_ref[...], preferred_element_type=jnp.float32))


def _msg(hsrc, attrp, w1, b1, tmat, w2s, bmat):
    return pl.pallas_call(
        _msg_body,
        grid=(GRID_E,),
        in_specs=[
            pl.BlockSpec((BE, 32), lambda j: (j, 0)),
            pl.BlockSpec((BE, 1), lambda j: (j, 0)),
            pl.BlockSpec((1, 16), lambda j: (0, 0)),
            pl.BlockSpec((1, 16), lambda j: (0, 0)),
            pl.BlockSpec((32, 512), lambda j: (0, 0)),
            pl.BlockSpec((512, 32), lambda j: (0, 0)),
            pl.BlockSpec((32, 32), lambda j: (0, 0)),
        ],
        out_specs=pl.BlockSpec((BE, 32), lambda j: (j, 0)),
        out_shape=jax.ShapeDtypeStruct((EP, 32), jnp.float32),
    )(hsrc, attrp, w1, b1, tmat, w2s, bmat)


def _prologue_body(bnd_ref, w1_ref, b1_ref, w2_ref, b2_ref, xt_ref, out_ref):
    h0 = jnp.maximum(bnd_ref[...] * w1_ref[...] + b1_ref[...], 0.0)
    h26 = jnp.maximum(
        jnp.dot(h0, w2_ref[...], preferred_element_type=jnp.float32)
        + b2_ref[...], 0.0)
    out_ref[...] = h26 + xt_ref[...]


def _prologue(bnd, fc1_w, fc1_b, fc2p, fc2bp, xt1):
    return pl.pallas_call(
        _prologue_body,
        out_shape=jax.ShapeDtypeStruct((NP, 32), jnp.float32),
    )(bnd, fc1_w, fc1_b, fc2p, fc2bp, xt1)


def _fin1_body(p_ref, cp_ref, h32_ref, root_ref, bias_ref, out_ref):
    cnt = cp_ref[0][:, 0:1] + cp_ref[1][:, 0:1]
    invc = 1.0 / jnp.maximum(cnt, 1.0)
    mean = (p_ref[0] + p_ref[1]) * invc
    out_ref[...] = jnp.maximum(
        mean + jnp.dot(h32_ref[...], root_ref[...],
                       preferred_element_type=jnp.float32) + bias_ref[...], 0.0)


def _fin1(p, cp, h32, root, bias):
    return pl.pallas_call(
        _fin1_body,
        out_shape=jax.ShapeDtypeStruct((NP, 32), jnp.float32),
    )(p, cp, h32, root, bias)


def _fin3_body(p_ref, cp_ref, hl1_ref, root_ref, bias_ref, fc3_ref, fc3b_ref,
               fc4_ref, fc4b_ref, xt_ref, h32_ref, yy_ref):
    cnt = cp_ref[0][:, 0:1] + cp_ref[1][:, 0:1]
    invc = 1.0 / jnp.maximum(cnt, 1.0)
    mean = (p_ref[0] + p_ref[1]) * invc
    h26 = jnp.maximum(
        mean + jnp.dot(hl1_ref[...], root_ref[...],
                       preferred_element_type=jnp.float32) + bias_ref[...], 0.0)
    z = jnp.maximum(
        jnp.dot(h26, fc3_ref[...], preferred_element_type=jnp.float32)
        + fc3b_ref[...], 0.0)
    yy_ref[...] = (jnp.dot(z, fc4_ref[...], preferred_element_type=jnp.float32)
                   + fc4b_ref[...])
    h32_ref[...] = h26 + xt_ref[...]


def _fin3(p, cp, hl1, root, bias, fc3p, fc3b, fc4, fc4b, xt):
    return pl.pallas_call(
        _fin3_body,
        out_shape=[
            jax.ShapeDtypeStruct((NP, 32), jnp.float32),
            jax.ShapeDtypeStruct((NP, 1), jnp.float32),
        ],
    )(p, cp, hl1, root, bias, fc3p, fc3b, fc4, fc4b, xt)


def kernel(x, t, edge_index, edge_attr, y, fc1_w, fc1_b, fc2_w, fc2_b,
           nn1_w1, nn1_b1, nn1_w2, nn1_b2, conv1_root, conv1_bias,
           nn3_w1, nn3_b1, nn3_w2, nn3_b2, conv3_root, conv3_bias,
           fc3_w, fc3_b, fc4_w, fc4_b):
    f32 = jnp.float32
    pad_e = EP - E

    src = edge_index[0].astype(jnp.int32)
    dst = edge_index[1].astype(jnp.int32)
    src2d = jnp.concatenate([src, jnp.zeros((pad_e,), jnp.int32)]).reshape(-1, CH)
    dst2d = jnp.concatenate(
        [dst, jnp.full((pad_e,), N, jnp.int32)]).reshape(-1, CH)
    attrp = jnp.pad(edge_attr, ((0, pad_e), (0, 0)))

    w2s1 = nn1_w2.reshape(16, 32, 32).transpose(1, 0, 2).reshape(512, 32)
    bm1 = nn1_b2.reshape(32, 32)
    w2s3 = jnp.pad(nn3_w2.reshape(16, 32, 26),
                   ((0, 0), (0, 0), (0, 6))).transpose(1, 0, 2).reshape(512, 32)
    bm3 = jnp.pad(nn3_b2.reshape(32, 26), ((0, 0), (0, 6)))
    tmat = jnp.kron(jnp.eye(32, dtype=jnp.bfloat16),
                    jnp.ones((1, 16), jnp.bfloat16))
    w2s1 = w2s1.astype(jnp.bfloat16)
    w2s3 = w2s3.astype(jnp.bfloat16)
    w1e1 = nn1_w1
    b1e1 = nn1_b1.reshape(1, 16)
    w1e3 = nn3_w1
    b1e3 = nn3_b1.reshape(1, 16)
    root1 = conv1_root
    bias1 = conv1_bias.reshape(1, 32)
    root3 = jnp.pad(conv3_root, ((0, 0), (0, 6)))
    bias3 = jnp.pad(conv3_bias, (0, 6)).reshape(1, 32)
    fc2p = jnp.pad(fc2_w, ((0, 0), (0, 6)))
    fc2bp = jnp.pad(fc2_b, (0, 6)).reshape(1, 32)
    fc3p = jnp.pad(fc3_w, ((0, 6), (0, 0)))
    fc3b = fc3_b.reshape(1, 32)
    fc4b = fc4_b.reshape(1, 1)

    xp = jnp.pad(x, ((0, NP - N), (0, 0)))
    zeros26 = jnp.zeros((NP, 26), f32)

    def xt_for(ti):
        return jnp.concatenate(
            [zeros26, xp, xp, xp, jnp.broadcast_to(ti, (NP, 3))], axis=1)

    zeros_np = jnp.zeros((NP, 32), f32)
    ones_ch = jnp.ones((CH, 32), f32)

    cp = _sc_count(ones_ch, dst2d, zeros_np)

    bnd = jnp.pad(y[0].reshape(-1, 1), ((0, NP - N), (0, 0)))
    h32 = _prologue(bnd, fc1_w, fc1_b.reshape(1, 32), fc2p, fc2bp, xt_for(t[1]))

    ys = []
    T = t.shape[0]
    for i in range(1, T):
        hs1 = _sc_gather(h32, src2d)
        m1 = _msg(hs1, attrp, w1e1, b1e1, tmat, w2s1, bm1)
        p1 = _sc_scatter(m1, dst2d, zeros_np)
        hl1 = _fin1(p1, cp, h32, root1, bias1)

        hs3 = _sc_gather(hl1, src2d)
        m3 = _msg(hs3, attrp, w1e3, b1e3, tmat, w2s3, bm3)
        p3 = _sc_scatter(m3, dst2d, zeros_np)
        xt_next = xt_for(t[i + 1]) if i + 1 < T else zeros_np
        h32, yy = _fin3(p3, cp, hl1, root3, bias3, fc3p, fc3b, fc4_w, fc4b,
                        xt_next)
        ys.append(yy[:N, 0])

    return jnp.concatenate([y[0]] + ys)
